# baseline (device time: 25522 ns/iter reference)
import jax
import jax.numpy as jnp
from jax import lax
from jax.experimental import pallas as pl
from jax.experimental.pallas import tpu as pltpu

N_DEV = 4


def kernel(x):
    m_per, n = x.shape
    m_half = m_per // 2

    def body(x_ref, out_ref, x_vmem, my_bf, local_sems, send_sems, recv_sems):
        my_pos = lax.axis_index("i")
        left = (my_pos - 1) % N_DEV
        right = (my_pos + 1) % N_DEV

        in_dma = pltpu.make_async_copy(x_ref, x_vmem, local_sems.at[0])
        in_dma.start()

        barrier_sem = pltpu.get_barrier_semaphore()
        for nbr in [left, right]:
            pl.semaphore_signal(
                barrier_sem, inc=1,
                device_id=(nbr,), device_id_type=pl.DeviceIdType.MESH,
            )
        pl.semaphore_wait(barrier_sem, 2)

        in_dma.wait()
        my_bf[:, :] = x_vmem[:, :].astype(jnp.bfloat16)

        def blk_top(b):
            return pl.ds(b * m_per, m_half)

        def blk_bot(b):
            return pl.ds(b * m_per + m_half, m_half)

        top = pl.ds(0, m_half)
        bot = pl.ds(m_half, m_half)

        def rdma(src, dst_sl, sem, dev):
            return pltpu.make_async_remote_copy(
                src_ref=src, dst_ref=out_ref.at[dst_sl, :],
                send_sem=send_sems.at[sem], recv_sem=recv_sems.at[sem],
                device_id=(dev,), device_id_type=pl.DeviceIdType.MESH,
            )

        a1 = rdma(my_bf.at[top, :], blk_top(my_pos), 0, right)
        b1 = rdma(my_bf.at[bot, :], blk_bot(my_pos), 1, left)
        a2 = rdma(my_bf.at[bot, :], blk_bot(my_pos), 2, right)
        b2 = rdma(my_bf.at[top, :], blk_top(my_pos), 3, left)
        a1.start()
        b1.start()
        a2.start()
        b2.start()

        own_dma = pltpu.make_async_copy(
            my_bf, out_ref.at[pl.ds(my_pos * m_per, m_per), :],
            local_sems.at[1],
        )
        own_dma.start()

        a1.wait_recv()
        a3 = rdma(out_ref.at[blk_top(left), :], blk_top(left), 4, right)
        a3.start()
        b1.wait_recv()
        b3 = rdma(out_ref.at[blk_bot(right), :], blk_bot(right), 5, left)
        b3.start()

        a2.wait_recv()
        b2.wait_recv()
        a3.wait_recv()
        b3.wait_recv()

        own_dma.wait()
        for op in (a1, b1, a2, b2, a3, b3):
            op.wait_send()

    return pl.pallas_call(
        body,
        out_shape=jax.ShapeDtypeStruct((N_DEV * m_per, n), jnp.bfloat16),
        in_specs=[pl.BlockSpec(memory_space=pl.ANY)],
        out_specs=pl.BlockSpec(memory_space=pl.ANY),
        scratch_shapes=[
            pltpu.VMEM((m_per, n), jnp.float32),
            pltpu.VMEM((m_per, n), jnp.bfloat16),
            pltpu.SemaphoreType.DMA((2,)),
            pltpu.SemaphoreType.DMA((6,)),
            pltpu.SemaphoreType.DMA((6,)),
        ],
        compiler_params=pltpu.CompilerParams(collective_id=0),
    )(x)
